# baseline (speedup 1.0000x reference)
import functools

import jax
import jax.numpy as jnp
from jax import lax
from jax.experimental import pallas as pl
from jax.experimental.pallas import tpu as pltpu
from jax.experimental.pallas import tpu_sc as plsc


def _node_proj(x, ws, wr):
    n, df = x.shape
    do = ws.shape[1]

    def body(x_ref, ws_ref, wr_ref, ns_ref, nr_ref):
        xv = x_ref[...]
        ns_ref[...] = jnp.dot(xv, ws_ref[...], preferred_element_type=jnp.float32)
        nr_ref[...] = jnp.dot(xv, wr_ref[...], preferred_element_type=jnp.float32)

    return pl.pallas_call(
        body,
        out_shape=(
            jax.ShapeDtypeStruct((n, do), jnp.float32),
            jax.ShapeDtypeStruct((n, do), jnp.float32),
        ),
    )(x, ws, wr)


def _sc_gather_sum(ns, nr, sidx, ridx, seg_off, n_edges):
    do = ns.shape[1]
    info = plsc.get_sparse_core_info()
    nc, nsub = info.num_cores, info.num_subcores
    nw = nc * nsub
    epw = n_edges // nw
    ch = 80
    nbuf = 5
    niter = epw // (ch * nbuf)
    mesh = plsc.VectorSubcoreMesh(core_axis_name="c", subcore_axis_name="s")

    @functools.partial(
        pl.kernel,
        out_type=jax.ShapeDtypeStruct((n_edges, do), jnp.float32),
        mesh=mesh,
        scratch_types=[
            pltpu.VMEM((epw,), jnp.int32),
            pltpu.VMEM((epw,), jnp.int32),
            pltpu.VMEM((nbuf, ch, do), jnp.float32),
            pltpu.SemaphoreType.DMA((nbuf,)),
            pltpu.SemaphoreType.DMA((nbuf,)),
            pltpu.SemaphoreType.DMA((nbuf,)),
        ],
    )
    def k(ns_hbm, nr_hbm, sidx_hbm, ridx_hbm, out_hbm, sidx_v, ridx_v, bufs,
          sema, semb, semw):
        wid = lax.axis_index("s") * nc + lax.axis_index("c")
        base = wid * epw
        pltpu.sync_copy(sidx_hbm.at[pl.ds(seg_off + base, epw)], sidx_v)
        pltpu.sync_copy(ridx_hbm.at[pl.ds(seg_off + base, epw)], ridx_v)

        def body(i, carry):
            off = i * (ch * nbuf)
            ga = []
            for j in range(nbuf):
                @pl.when(i > 0)
                def _(j=j):
                    pltpu.make_async_copy(
                        bufs.at[j],
                        out_hbm.at[pl.ds(base + off + j * ch, ch), :],
                        semw.at[j]).wait()
                ga.append(pltpu.async_copy(
                    ns_hbm.at[sidx_v.at[pl.ds(off + j * ch, ch)]],
                    bufs.at[j], sema.at[j]))
            gb = []
            for j in range(nbuf):
                ga[j].wait()
                gb.append(pltpu.async_copy(
                    nr_hbm.at[ridx_v.at[pl.ds(off + j * ch, ch)]],
                    bufs.at[j], semb.at[j], add=True))
            for j in range(nbuf):
                gb[j].wait()
                pltpu.async_copy(
                    bufs.at[j], out_hbm.at[pl.ds(base + off + j * ch, ch), :],
                    semw.at[j])
            return carry

        lax.fori_loop(0, niter, body, 0)
        lastoff = (niter - 1) * (ch * nbuf)
        for j in range(nbuf):
            pltpu.make_async_copy(
                bufs.at[j],
                out_hbm.at[pl.ds(base + lastoff + j * ch, ch), :],
                semw.at[j]).wait()

    return k(ns, nr, sidx, ridx)


def _edge_mlp_seg(prev, gsum, ea_t, we, b2d, e, blk0, be):
    de = ea_t.shape[0]
    do = we.shape[1]
    eseg = gsum.shape[0]

    def body(*refs):
        g_ref, eat_ref, we_ref, b_ref, o_ref = refs[-5:]
        prod = lax.dot_general(
            eat_ref[...], we_ref[...],
            dimension_numbers=(((0,), (0,)), ((), ())),
            preferred_element_type=jnp.float32,
        )
        o_ref[...] = g_ref[...] + prod + b_ref[...]

    specs = [
        pl.BlockSpec((be, do), lambda i: (i, 0)),
        pl.BlockSpec((de, be), lambda i: (0, i + blk0)),
        pl.BlockSpec((de, do), lambda i: (0, 0)),
        pl.BlockSpec((1, do), lambda i: (0, 0)),
    ]
    args = (gsum, ea_t, we, b2d)
    aliases = {}
    if prev is not None:
        specs = [pl.BlockSpec(memory_space=pl.ANY)] + specs
        args = (prev,) + args
        aliases = {0: 0}
    return pl.pallas_call(
        body,
        grid=(eseg // be,),
        in_specs=specs,
        out_specs=pl.BlockSpec((be, do), lambda i: (i + blk0, 0)),
        out_shape=jax.ShapeDtypeStruct((e, do), jnp.float32),
        input_output_aliases=aliases,
    )(*args)


def kernel(x, edge_index, edge_attr, W, b):
    n, df = x.shape
    e, de = edge_attr.shape
    do = W.shape[1]
    senders = edge_index[0].astype(jnp.int32)
    receivers = edge_index[1].astype(jnp.int32)
    ws = W[:df]
    wr = W[df:2 * df]
    we = W[2 * df:]
    b2d = b.reshape(1, do)
    ea_t = edge_attr.T
    ns, nr = _node_proj(x, ws, wr)
    be = 16000
    segs = (e // 5, e * 2 // 5, e * 2 // 5)
    out = None
    off = 0
    for eseg in segs:
        gsum = _sc_gather_sum(ns, nr, senders, receivers, off, eseg)
        out = _edge_mlp_seg(out, gsum, ea_t, we, b2d, e, off // be, be)
        off += eseg
    return out


# speedup vs baseline: 1.0285x; 1.0285x over previous
import functools

import jax
import jax.numpy as jnp
from jax import lax
from jax.experimental import pallas as pl
from jax.experimental.pallas import tpu as pltpu
from jax.experimental.pallas import tpu_sc as plsc


def _node_proj(x, ws, wr):
    n, df = x.shape
    do = ws.shape[1]

    def body(x_ref, ws_ref, wr_ref, ns_ref, nr_ref):
        xv = x_ref[...]
        ns_ref[...] = jnp.dot(xv, ws_ref[...], preferred_element_type=jnp.float32)
        nr_ref[...] = jnp.dot(xv, wr_ref[...], preferred_element_type=jnp.float32)

    return pl.pallas_call(
        body,
        out_shape=(
            jax.ShapeDtypeStruct((n, do), jnp.float32),
            jax.ShapeDtypeStruct((n, do), jnp.float32),
        ),
    )(x, ws, wr)


def _sc_gather_sum(ns, nr, sidx, ridx, n_edges):
    do = ns.shape[1]
    info = plsc.get_sparse_core_info()
    nc, nsub = info.num_cores, info.num_subcores
    nw = nc * nsub
    epw = n_edges // nw
    ch = 40
    nbuf = 10
    niter = epw // (ch * nbuf)
    mesh = plsc.VectorSubcoreMesh(core_axis_name="c", subcore_axis_name="s")

    @functools.partial(
        pl.kernel,
        out_type=jax.ShapeDtypeStruct((n_edges, do), jnp.float32),
        mesh=mesh,
        scratch_types=[
            pltpu.VMEM((epw,), jnp.int32),
            pltpu.VMEM((epw,), jnp.int32),
            pltpu.VMEM((nbuf, ch, do), jnp.float32),
            pltpu.SemaphoreType.DMA((nbuf,)),
            pltpu.SemaphoreType.DMA((nbuf,)),
            pltpu.SemaphoreType.DMA((nbuf,)),
        ],
    )
    def k(ns_hbm, nr_hbm, sidx_hbm, ridx_hbm, out_hbm, sidx_v, ridx_v, bufs,
          sema, semb, semw):
        wid = lax.axis_index("s") * nc + lax.axis_index("c")
        base = wid * epw
        pltpu.sync_copy(sidx_hbm.at[pl.ds(base, epw)], sidx_v)
        pltpu.sync_copy(ridx_hbm.at[pl.ds(base, epw)], ridx_v)

        def body(i, carry):
            off = i * (ch * nbuf)
            ga = []
            for j in range(nbuf):
                @pl.when(i > 0)
                def _(j=j):
                    pltpu.make_async_copy(
                        bufs.at[j],
                        out_hbm.at[pl.ds(base + off + j * ch, ch), :],
                        semw.at[j]).wait()
                ga.append(pltpu.async_copy(
                    ns_hbm.at[sidx_v.at[pl.ds(off + j * ch, ch)]],
                    bufs.at[j], sema.at[j]))
            gb = []
            for j in range(nbuf):
                ga[j].wait()
                gb.append(pltpu.async_copy(
                    nr_hbm.at[ridx_v.at[pl.ds(off + j * ch, ch)]],
                    bufs.at[j], semb.at[j], add=True))
            for j in range(nbuf):
                gb[j].wait()
                pltpu.async_copy(
                    bufs.at[j], out_hbm.at[pl.ds(base + off + j * ch, ch), :],
                    semw.at[j])
            return carry

        lax.fori_loop(0, niter, body, 0)
        lastoff = (niter - 1) * (ch * nbuf)
        for j in range(nbuf):
            pltpu.make_async_copy(
                bufs.at[j],
                out_hbm.at[pl.ds(base + lastoff + j * ch, ch), :],
                semw.at[j]).wait()

    return k(ns, nr, sidx, ridx)


def _edge_mlp(gsum, ea_t, we, b2d):
    de, e = ea_t.shape
    do = we.shape[1]
    be = 16000
    grid = (e // be,)

    def body(g_ref, eat_ref, we_ref, b_ref, o_ref):
        prod = lax.dot_general(
            eat_ref[...], we_ref[...],
            dimension_numbers=(((0,), (0,)), ((), ())),
            preferred_element_type=jnp.float32,
        )
        o_ref[...] = g_ref[...] + prod + b_ref[...]

    return pl.pallas_call(
        body,
        grid=grid,
        in_specs=[
            pl.BlockSpec((be, do), lambda i: (i, 0)),
            pl.BlockSpec((de, be), lambda i: (0, i)),
            pl.BlockSpec((de, do), lambda i: (0, 0)),
            pl.BlockSpec((1, do), lambda i: (0, 0)),
        ],
        out_specs=pl.BlockSpec((be, do), lambda i: (i, 0)),
        out_shape=jax.ShapeDtypeStruct((e, do), jnp.float32),
    )(gsum, ea_t, we, b2d)


def kernel(x, edge_index, edge_attr, W, b):
    n, df = x.shape
    e, de = edge_attr.shape
    do = W.shape[1]
    senders = edge_index[0].astype(jnp.int32)
    receivers = edge_index[1].astype(jnp.int32)
    ws = W[:df]
    wr = W[df:2 * df]
    we = W[2 * df:]
    ns, nr = _node_proj(x, ws, wr)
    gsum = _sc_gather_sum(ns, nr, senders, receivers, e)
    return _edge_mlp(gsum, edge_attr.T, we, b.reshape(1, do))
